# Initial kernel scaffold; baseline (speedup 1.0000x reference)
import jax
import jax.numpy as jnp
from jax.experimental import pallas as pl


def kernel(adj, semantics, attention_masks, W):
    raise NotImplementedError("write your pallas kernel here")


# speedup vs baseline: 1089.7499x; 1089.7499x over previous
import jax
import jax.numpy as jnp
from jax.experimental import pallas as pl


def _mplayer_kernel(s0_ref, w_ref, adj_ref, out_ref):
    s0 = s0_ref[...]
    w = w_ref[...]
    adj = adj_ref[...]
    h = jnp.dot(s0, w, preferred_element_type=jnp.float32)
    h_sum = jax.lax.dot_general(
        adj, h, (((0,), (0,)), ((), ())), preferred_element_type=jnp.float32
    )
    counts = jnp.sum(adj, axis=0)
    h_mean = h_sum / jnp.maximum(counts, 1.0)[:, None]
    inv_sqrt2 = 0.7071067811865476
    out_ref[...] = 0.5 * h_mean * (1.0 + jax.lax.erf(h_mean * inv_sqrt2))


def kernel(adj, semantics, attention_masks, W):
    n = adj.shape[0]
    hidden = W.shape[0]
    s0 = semantics[:, 0, :]
    return pl.pallas_call(
        _mplayer_kernel,
        out_shape=jax.ShapeDtypeStruct((n, hidden), jnp.float32),
    )(s0, W, adj)
